# baseline (speedup 1.0000x reference)
import functools

import jax
import jax.numpy as jnp
from jax import lax
from jax.experimental import pallas as pl
from jax.experimental.pallas import tpu as pltpu
from jax.experimental.pallas import tpu_sc as plsc

N = 100000
D = 32
E = 1600000
LAMBDA = 0.01

NC = 2
NS = 16
NW = NC * NS
EPW = E // NW
BLK = 400
NBLK = EPW // BLK
CHUNK = 80
NCHUNK = BLK // CHUNK
NGRP = BLK // 16


def _sqrt16(n):
    n = jnp.maximum(n, jnp.full((16,), 1e-30, jnp.float32))
    i = plsc.bitcast(n, jnp.int32)
    i = jnp.int32(0x5F3759DF) - lax.shift_right_logical(i, 1)
    x = plsc.bitcast(i, jnp.float32)
    for _ in range(3):
        x = x * (1.5 - 0.5 * n * x * x)
    return n * x


def _make_sc_kernel():
    mesh = plsc.VectorSubcoreMesh(core_axis_name="c", subcore_axis_name="s")

    @functools.partial(
        pl.kernel,
        mesh=mesh,
        compiler_params=pltpu.CompilerParams(
            needs_layout_passes=False, use_tc_tiling_on_sc=False),
        out_type=(
            jax.ShapeDtypeStruct((E,), jnp.float32),
            jax.ShapeDtypeStruct((2, NW, 16), jnp.float32),
        ),
        scratch_types=[
            pltpu.VMEM((2, BLK), jnp.int32),
            pltpu.VMEM((2, BLK), jnp.int32),
            pltpu.VMEM((2, BLK), jnp.float32),
            pltpu.VMEM((2 * BLK, D), jnp.float32),
            pltpu.VMEM((2 * BLK, D), jnp.float32),
            pltpu.VMEM((2, BLK), jnp.float32),
            pltpu.VMEM((2, 16), jnp.float32),
            pltpu.SemaphoreType.DMA,
            pltpu.SemaphoreType.DMA,
            pltpu.SemaphoreType.DMA,
            pltpu.SemaphoreType.DMA,
            pltpu.SemaphoreType.DMA,
        ],
    )
    def sc_kernel(src_hbm, dst_hbm, lab_hbm, tab_hbm, y_hbm, part_hbm,
                  idx_s, idx_d, lab_v, rows_s, rows_d, y_v, acc_v,
                  sem_s, sem_d, sem_i, sem_y0, sem_y1):
        wid = lax.axis_index("s") * NC + lax.axis_index("c")
        base = wid * EPW
        iota = lax.iota(jnp.int32, 16)
        zeros = jnp.zeros((16,), jnp.float32)

        def fire_gathers(par):
            for j in range(NCHUNK):
                sl = pl.ds(j * CHUNK, CHUNK)
                roff = pl.ds(par * BLK + j * CHUNK, CHUNK)
                pltpu.async_copy(
                    tab_hbm.at[idx_s.at[par, sl]], rows_s.at[roff], sem_s)
                pltpu.async_copy(
                    tab_hbm.at[idx_d.at[par, sl]], rows_d.at[roff], sem_d)

        def fire_idx(par, k):
            off = pl.ds(base + k * BLK, BLK)
            pltpu.async_copy(src_hbm.at[off], idx_s.at[par], sem_i)
            pltpu.async_copy(dst_hbm.at[off], idx_d.at[par], sem_i)

        def drain_rows(par):
            del par
            pltpu.make_async_copy(
                tab_hbm.at[pl.ds(0, BLK)], rows_s.at[pl.ds(0, BLK)],
                sem_s).wait()
            pltpu.make_async_copy(
                tab_hbm.at[pl.ds(0, BLK)], rows_d.at[pl.ds(0, BLK)],
                sem_d).wait()

        def drain_idx():
            pltpu.make_async_copy(
                src_hbm.at[pl.ds(0, BLK)], idx_s.at[0], sem_i).wait()
            pltpu.make_async_copy(
                dst_hbm.at[pl.ds(0, BLK)], idx_d.at[0], sem_i).wait()
            pltpu.make_async_copy(
                lab_hbm.at[pl.ds(0, BLK)], lab_v.at[0], sem_i).wait()

        pltpu.sync_copy(src_hbm.at[pl.ds(base, BLK)], idx_s.at[0])
        pltpu.sync_copy(dst_hbm.at[pl.ds(base, BLK)], idx_d.at[0])
        pltpu.sync_copy(lab_hbm.at[pl.ds(base, BLK)], lab_v.at[0])
        fire_gathers(0)
        fire_idx(1, 1)
        pltpu.async_copy(lab_hbm.at[pl.ds(base + BLK, BLK)], lab_v.at[1],
                         sem_i)

        def block_body(k, acc):
            a_sse, a_nrm = acc
            par = k & 1
            nxt = 1 - par
            off = base + k * BLK

            drain_rows(par)

            @pl.when(k < NBLK - 1)
            def _():
                drain_idx()
                fire_gathers(nxt)

            @pl.when(k < NBLK - 2)
            def _():
                fire_idx(par, k + 2)

            @pl.when(jnp.logical_and(k >= 2, par == 0))
            def _():
                pltpu.make_async_copy(
                    y_v.at[0], y_hbm.at[pl.ds(base, BLK)], sem_y0).wait()

            @pl.when(jnp.logical_and(k >= 2, par == 1))
            def _():
                pltpu.make_async_copy(
                    y_v.at[1], y_hbm.at[pl.ds(base, BLK)], sem_y1).wait()

            def grp_body(g, acc):
                a_sse, a_nrm = acc
                row0 = g * 16
                ids = par * BLK + row0 + iota
                y = zeros
                n2 = zeros
                for d in range(D):
                    col = jnp.bitwise_and(iota + d, D - 1)
                    a = plsc.load_gather(rows_s, [ids, col])
                    b = plsc.load_gather(rows_d, [ids, col])
                    y = y + a * b
                    n2 = n2 + a * a
                lab = lab_v[par, pl.ds(row0, 16)]
                y_v[par, pl.ds(row0, 16)] = y
                r = lab - y
                return (a_sse + r * r, a_nrm + _sqrt16(n2))

            def grp5_body(g5, acc):
                for u in range(5):
                    acc = grp_body(g5 * 5 + u, acc)
                return acc

            acc = lax.fori_loop(0, NGRP // 5, grp5_body, (a_sse, a_nrm))

            @pl.when(par == 0)
            def _():
                pltpu.async_copy(y_v.at[0], y_hbm.at[pl.ds(off, BLK)], sem_y0)

            @pl.when(par == 1)
            def _():
                pltpu.async_copy(y_v.at[1], y_hbm.at[pl.ds(off, BLK)], sem_y1)

            @pl.when(k < NBLK - 2)
            def _():
                pltpu.async_copy(lab_hbm.at[pl.ds(off + 2 * BLK, BLK)],
                                 lab_v.at[par], sem_i)

            return acc

        a_sse, a_nrm = lax.fori_loop(0, NBLK, block_body, (zeros, zeros))

        pltpu.make_async_copy(
            y_v.at[0], y_hbm.at[pl.ds(base, BLK)], sem_y0).wait()
        pltpu.make_async_copy(
            y_v.at[1], y_hbm.at[pl.ds(base, BLK)], sem_y1).wait()

        acc_v[0, :] = a_sse
        acc_v[1, :] = a_nrm
        pltpu.sync_copy(acc_v.at[0], part_hbm.at[0, wid])
        pltpu.sync_copy(acc_v.at[1], part_hbm.at[1, wid])

    return sc_kernel


def _loss_body(p_ref, out_ref):
    s_sse = jnp.sum(p_ref[0:1, :])
    s_nrm = jnp.sum(p_ref[1:2, :])
    l = jnp.float32(E) * 0.5 * s_sse + jnp.float32(LAMBDA / 2.0) * s_nrm
    out_ref[...] = l.reshape(1, 1)


def kernel(edges, labels, embedding_state):
    edges = edges.astype(jnp.int32)
    src = edges[:, 0]
    dst = edges[:, 1]
    labels = labels.astype(jnp.float32)
    table = embedding_state.astype(jnp.float32)

    y, partials = _make_sc_kernel()(src, dst, labels, table)

    l_out = pl.pallas_call(
        _loss_body,
        out_shape=jax.ShapeDtypeStruct((1, 1), jnp.float32),
    )(partials.reshape(2, NW * 16))
    return (l_out[0, 0], y)


# speedup vs baseline: 1.2568x; 1.2568x over previous
import functools

import jax
import jax.numpy as jnp
from jax import lax
from jax.experimental import pallas as pl
from jax.experimental.pallas import tpu as pltpu
from jax.experimental.pallas import tpu_sc as plsc

N = 100000
D = 32
E = 1600000
LAMBDA = 0.01

NC = 2
NS = 16
NW = NC * NS
EPW = E // NW
BLK = 400
NBLK = EPW // BLK
CHUNK = 80
NCHUNK = BLK // CHUNK
NGRP = BLK // 16


def _sqrt16(n):
    n = jnp.maximum(n, jnp.full((16,), 1e-30, jnp.float32))
    i = plsc.bitcast(n, jnp.int32)
    i = jnp.int32(0x5F3759DF) - lax.shift_right_logical(i, 1)
    x = plsc.bitcast(i, jnp.float32)
    for _ in range(3):
        x = x * (1.5 - 0.5 * n * x * x)
    return n * x


def _make_sc_kernel():
    mesh = plsc.VectorSubcoreMesh(core_axis_name="c", subcore_axis_name="s")

    @functools.partial(
        pl.kernel,
        mesh=mesh,
        compiler_params=pltpu.CompilerParams(
            needs_layout_passes=False, use_tc_tiling_on_sc=False),
        out_type=(
            jax.ShapeDtypeStruct((E,), jnp.float32),
            jax.ShapeDtypeStruct((2, NW, 16), jnp.float32),
        ),
        scratch_types=[
            pltpu.VMEM((2, BLK), jnp.int32),
            pltpu.VMEM((2, BLK), jnp.int32),
            pltpu.VMEM((2, BLK), jnp.float32),
            pltpu.VMEM((2 * BLK, D), jnp.float32),
            pltpu.VMEM((2 * BLK, D), jnp.float32),
            pltpu.VMEM((2, BLK), jnp.float32),
            pltpu.VMEM((2, 16), jnp.float32),
            pltpu.SemaphoreType.DMA,
            pltpu.SemaphoreType.DMA,
            pltpu.SemaphoreType.DMA,
            pltpu.SemaphoreType.DMA,
            pltpu.SemaphoreType.DMA,
        ],
    )
    def sc_kernel(src_hbm, dst_hbm, lab_hbm, tab_hbm, y_hbm, part_hbm,
                  idx_s, idx_d, lab_v, rows_s, rows_d, y_v, acc_v,
                  sem_s, sem_d, sem_i, sem_y0, sem_y1):
        wid = lax.axis_index("s") * NC + lax.axis_index("c")
        base = wid * EPW
        iota = lax.iota(jnp.int32, 16)
        zeros = jnp.zeros((16,), jnp.float32)

        def fire_gathers(par):
            for j in range(NCHUNK):
                sl = pl.ds(j * CHUNK, CHUNK)
                roff = pl.ds(par * BLK + j * CHUNK, CHUNK)
                pltpu.async_copy(
                    tab_hbm.at[idx_s.at[par, sl]], rows_s.at[roff], sem_s)
                pltpu.async_copy(
                    tab_hbm.at[idx_d.at[par, sl]], rows_d.at[roff], sem_d)

        def fire_idx(par, k):
            off = pl.ds(base + k * BLK, BLK)
            pltpu.async_copy(src_hbm.at[off], idx_s.at[par], sem_i)
            pltpu.async_copy(dst_hbm.at[off], idx_d.at[par], sem_i)

        def drain_rows(par):
            del par
            pltpu.make_async_copy(
                tab_hbm.at[pl.ds(0, BLK)], rows_s.at[pl.ds(0, BLK)],
                sem_s).wait()
            pltpu.make_async_copy(
                tab_hbm.at[pl.ds(0, BLK)], rows_d.at[pl.ds(0, BLK)],
                sem_d).wait()

        def drain_idx():
            pltpu.make_async_copy(
                src_hbm.at[pl.ds(0, BLK)], idx_s.at[0], sem_i).wait()
            pltpu.make_async_copy(
                dst_hbm.at[pl.ds(0, BLK)], idx_d.at[0], sem_i).wait()
            pltpu.make_async_copy(
                lab_hbm.at[pl.ds(0, BLK)], lab_v.at[0], sem_i).wait()

        pltpu.sync_copy(src_hbm.at[pl.ds(base, BLK)], idx_s.at[0])
        pltpu.sync_copy(dst_hbm.at[pl.ds(base, BLK)], idx_d.at[0])
        pltpu.sync_copy(lab_hbm.at[pl.ds(base, BLK)], lab_v.at[0])
        fire_gathers(0)
        fire_idx(1, 1)
        pltpu.async_copy(lab_hbm.at[pl.ds(base + BLK, BLK)], lab_v.at[1],
                         sem_i)

        def block_body(k, acc):
            a_sse, a_nrm = acc
            par = k & 1
            nxt = 1 - par
            off = base + k * BLK

            drain_rows(par)

            @pl.when(k < NBLK - 1)
            def _():
                drain_idx()
                fire_gathers(nxt)

            @pl.when(k < NBLK - 2)
            def _():
                fire_idx(par, k + 2)

            @pl.when(jnp.logical_and(k >= 2, par == 0))
            def _():
                pltpu.make_async_copy(
                    y_v.at[0], y_hbm.at[pl.ds(base, BLK)], sem_y0).wait()

            @pl.when(jnp.logical_and(k >= 2, par == 1))
            def _():
                pltpu.make_async_copy(
                    y_v.at[1], y_hbm.at[pl.ds(base, BLK)], sem_y1).wait()

            def grp_body(g, acc):
                a_sse, a_nrm = acc
                row0 = g * 16
                ids = par * BLK + row0 + iota
                y = zeros
                n2 = zeros
                for d in range(D):
                    col = jnp.bitwise_and(iota + d, D - 1)
                    a = plsc.load_gather(rows_s, [ids, col])
                    b = plsc.load_gather(rows_d, [ids, col])
                    y = y + a * b
                    n2 = n2 + a * a
                lab = lab_v[par, pl.ds(row0, 16)]
                y_v[par, pl.ds(row0, 16)] = y
                r = lab - y
                return (a_sse + r * r, a_nrm + _sqrt16(n2))

            acc = lax.fori_loop(0, NGRP, grp_body, (a_sse, a_nrm))

            @pl.when(par == 0)
            def _():
                pltpu.async_copy(y_v.at[0], y_hbm.at[pl.ds(off, BLK)], sem_y0)

            @pl.when(par == 1)
            def _():
                pltpu.async_copy(y_v.at[1], y_hbm.at[pl.ds(off, BLK)], sem_y1)

            @pl.when(k < NBLK - 2)
            def _():
                pltpu.async_copy(lab_hbm.at[pl.ds(off + 2 * BLK, BLK)],
                                 lab_v.at[par], sem_i)

            return acc

        a_sse, a_nrm = lax.fori_loop(0, NBLK, block_body, (zeros, zeros))

        pltpu.make_async_copy(
            y_v.at[0], y_hbm.at[pl.ds(base, BLK)], sem_y0).wait()
        pltpu.make_async_copy(
            y_v.at[1], y_hbm.at[pl.ds(base, BLK)], sem_y1).wait()

        acc_v[0, :] = a_sse
        acc_v[1, :] = a_nrm
        pltpu.sync_copy(acc_v.at[0], part_hbm.at[0, wid])
        pltpu.sync_copy(acc_v.at[1], part_hbm.at[1, wid])

    return sc_kernel


def _loss_body(p_ref, out_ref):
    s_sse = jnp.sum(p_ref[0:1, :])
    s_nrm = jnp.sum(p_ref[1:2, :])
    l = jnp.float32(E) * 0.5 * s_sse + jnp.float32(LAMBDA / 2.0) * s_nrm
    out_ref[...] = l.reshape(1, 1)


def kernel(edges, labels, embedding_state):
    edges = edges.astype(jnp.int32)
    src = edges[:, 0]
    dst = edges[:, 1]
    labels = labels.astype(jnp.float32)
    table = embedding_state.astype(jnp.float32)

    y, partials = _make_sc_kernel()(src, dst, labels, table)

    l_out = pl.pallas_call(
        _loss_body,
        out_shape=jax.ShapeDtypeStruct((1, 1), jnp.float32),
    )(partials.reshape(2, NW * 16))
    return (l_out[0, 0], y)


# speedup vs baseline: 1.2652x; 1.0066x over previous
import functools

import jax
import jax.numpy as jnp
from jax import lax
from jax.experimental import pallas as pl
from jax.experimental.pallas import tpu as pltpu
from jax.experimental.pallas import tpu_sc as plsc

N = 100000
D = 32
E = 1600000
LAMBDA = 0.01

NC = 2
NS = 16
NW = NC * NS
EPW = E // NW
BLK = 400
NBLK = EPW // BLK
CHUNK = 80
NCHUNK = BLK // CHUNK
NGRP = BLK // 16


def _sqrt16(n):
    n = jnp.maximum(n, jnp.full((16,), 1e-30, jnp.float32))
    i = plsc.bitcast(n, jnp.int32)
    i = jnp.int32(0x5F3759DF) - lax.shift_right_logical(i, 1)
    x = plsc.bitcast(i, jnp.float32)
    for _ in range(3):
        x = x * (1.5 - 0.5 * n * x * x)
    return n * x


def _make_sc_kernel():
    mesh = plsc.VectorSubcoreMesh(core_axis_name="c", subcore_axis_name="s")

    @functools.partial(
        pl.kernel,
        mesh=mesh,
        compiler_params=pltpu.CompilerParams(
            needs_layout_passes=False, use_tc_tiling_on_sc=False),
        out_type=(
            jax.ShapeDtypeStruct((E,), jnp.float32),
            jax.ShapeDtypeStruct((2, NW, 16), jnp.float32),
        ),
        scratch_types=[
            pltpu.VMEM((2, BLK), jnp.int32),
            pltpu.VMEM((2, BLK), jnp.int32),
            pltpu.VMEM((2, BLK), jnp.float32),
            pltpu.VMEM((2 * BLK, D), jnp.float32),
            pltpu.VMEM((2 * BLK, D), jnp.float32),
            pltpu.VMEM((2, BLK), jnp.float32),
            pltpu.VMEM((2, 16), jnp.float32),
            pltpu.SemaphoreType.DMA,
            pltpu.SemaphoreType.DMA,
            pltpu.SemaphoreType.DMA,
            pltpu.SemaphoreType.DMA,
            pltpu.SemaphoreType.DMA,
        ],
    )
    def sc_kernel(src_hbm, dst_hbm, lab_hbm, tab_hbm, y_hbm, part_hbm,
                  idx_s, idx_d, lab_v, rows_s, rows_d, y_v, acc_v,
                  sem_s, sem_d, sem_i, sem_y0, sem_y1):
        wid = lax.axis_index("s") * NC + lax.axis_index("c")
        base = wid * EPW
        iota = lax.iota(jnp.int32, 16)
        zeros = jnp.zeros((16,), jnp.float32)
        zero16 = jnp.zeros((16,), jnp.int32)

        def fire_gathers(par):
            for j in range(NCHUNK):
                sl = pl.ds(j * CHUNK, CHUNK)
                roff = pl.ds(par * BLK + j * CHUNK, CHUNK)
                pltpu.async_copy(
                    tab_hbm.at[idx_s.at[par, sl]], rows_s.at[roff], sem_s)
                pltpu.async_copy(
                    tab_hbm.at[idx_d.at[par, sl]], rows_d.at[roff], sem_d)

        def fire_idx(par, k):
            off = pl.ds(base + k * BLK, BLK)
            pltpu.async_copy(src_hbm.at[off], idx_s.at[par], sem_i)
            pltpu.async_copy(dst_hbm.at[off], idx_d.at[par], sem_i)

        def drain_rows(par):
            del par
            pltpu.make_async_copy(
                tab_hbm.at[pl.ds(0, BLK)], rows_s.at[pl.ds(0, BLK)],
                sem_s).wait()
            pltpu.make_async_copy(
                tab_hbm.at[pl.ds(0, BLK)], rows_d.at[pl.ds(0, BLK)],
                sem_d).wait()

        def drain_idx():
            pltpu.make_async_copy(
                src_hbm.at[pl.ds(0, BLK)], idx_s.at[0], sem_i).wait()
            pltpu.make_async_copy(
                dst_hbm.at[pl.ds(0, BLK)], idx_d.at[0], sem_i).wait()
            pltpu.make_async_copy(
                lab_hbm.at[pl.ds(0, BLK)], lab_v.at[0], sem_i).wait()

        pltpu.sync_copy(src_hbm.at[pl.ds(base, BLK)], idx_s.at[0])
        pltpu.sync_copy(dst_hbm.at[pl.ds(base, BLK)], idx_d.at[0])
        pltpu.sync_copy(lab_hbm.at[pl.ds(base, BLK)], lab_v.at[0])
        fire_gathers(0)
        fire_idx(1, 1)
        pltpu.async_copy(lab_hbm.at[pl.ds(base + BLK, BLK)], lab_v.at[1],
                         sem_i)

        def block_body(k, acc):
            a_sse, a_nrm = acc
            par = k & 1
            nxt = 1 - par
            off = base + k * BLK

            drain_rows(par)

            @pl.when(k < NBLK - 1)
            def _():
                drain_idx()
                fire_gathers(nxt)

            @pl.when(k < NBLK - 2)
            def _():
                fire_idx(par, k + 2)

            @pl.when(jnp.logical_and(k >= 2, par == 0))
            def _():
                pltpu.make_async_copy(
                    y_v.at[0], y_hbm.at[pl.ds(base, BLK)], sem_y0).wait()

            @pl.when(jnp.logical_and(k >= 2, par == 1))
            def _():
                pltpu.make_async_copy(
                    y_v.at[1], y_hbm.at[pl.ds(base, BLK)], sem_y1).wait()

            def grp_body(g, acc):
                a_sse, a_nrm = acc
                row0 = g * 16
                ids = par * BLK + row0 + iota
                basev = ids * D + iota
                y = zeros
                n2 = zeros
                for d in range(D):
                    addr = jnp.bitwise_xor(basev, d)
                    a = plsc.load_gather(rows_s, [zero16, addr])
                    b = plsc.load_gather(rows_d, [zero16, addr])
                    y = y + a * b
                    n2 = n2 + a * a
                lab = lab_v[par, pl.ds(row0, 16)]
                y_v[par, pl.ds(row0, 16)] = y
                r = lab - y
                return (a_sse + r * r, a_nrm + _sqrt16(n2))

            acc = lax.fori_loop(0, NGRP, grp_body, (a_sse, a_nrm))

            @pl.when(par == 0)
            def _():
                pltpu.async_copy(y_v.at[0], y_hbm.at[pl.ds(off, BLK)], sem_y0)

            @pl.when(par == 1)
            def _():
                pltpu.async_copy(y_v.at[1], y_hbm.at[pl.ds(off, BLK)], sem_y1)

            @pl.when(k < NBLK - 2)
            def _():
                pltpu.async_copy(lab_hbm.at[pl.ds(off + 2 * BLK, BLK)],
                                 lab_v.at[par], sem_i)

            return acc

        a_sse, a_nrm = lax.fori_loop(0, NBLK, block_body, (zeros, zeros))

        pltpu.make_async_copy(
            y_v.at[0], y_hbm.at[pl.ds(base, BLK)], sem_y0).wait()
        pltpu.make_async_copy(
            y_v.at[1], y_hbm.at[pl.ds(base, BLK)], sem_y1).wait()

        acc_v[0, :] = a_sse
        acc_v[1, :] = a_nrm
        pltpu.sync_copy(acc_v.at[0], part_hbm.at[0, wid])
        pltpu.sync_copy(acc_v.at[1], part_hbm.at[1, wid])

    return sc_kernel


def _loss_body(p_ref, out_ref):
    s_sse = jnp.sum(p_ref[0:1, :])
    s_nrm = jnp.sum(p_ref[1:2, :])
    l = jnp.float32(E) * 0.5 * s_sse + jnp.float32(LAMBDA / 2.0) * s_nrm
    out_ref[...] = l.reshape(1, 1)


def kernel(edges, labels, embedding_state):
    edges = edges.astype(jnp.int32)
    src = edges[:, 0]
    dst = edges[:, 1]
    labels = labels.astype(jnp.float32)
    table = embedding_state.astype(jnp.float32)

    y, partials = _make_sc_kernel()(src, dst, labels, table)

    l_out = pl.pallas_call(
        _loss_body,
        out_shape=jax.ShapeDtypeStruct((1, 1), jnp.float32),
    )(partials.reshape(2, NW * 16))
    return (l_out[0, 0], y)


# speedup vs baseline: 1.2701x; 1.0039x over previous
import functools

import jax
import jax.numpy as jnp
from jax import lax
from jax.experimental import pallas as pl
from jax.experimental.pallas import tpu as pltpu
from jax.experimental.pallas import tpu_sc as plsc

N = 100000
D = 32
E = 1600000
LAMBDA = 0.01

NC = 2
NS = 16
NW = NC * NS
EPW = E // NW
BLK = 400
NBLK = EPW // BLK
CHUNK = 80
NCHUNK = BLK // CHUNK
NGRP = BLK // 16


def _sqrt16(n):
    n = jnp.maximum(n, jnp.full((16,), 1e-30, jnp.float32))
    i = plsc.bitcast(n, jnp.int32)
    i = jnp.int32(0x5F3759DF) - lax.shift_right_logical(i, 1)
    x = plsc.bitcast(i, jnp.float32)
    for _ in range(3):
        x = x * (1.5 - 0.5 * n * x * x)
    return n * x


def _make_sc_kernel():
    mesh = plsc.VectorSubcoreMesh(core_axis_name="c", subcore_axis_name="s")

    @functools.partial(
        pl.kernel,
        mesh=mesh,
        compiler_params=pltpu.CompilerParams(
            needs_layout_passes=False, use_tc_tiling_on_sc=False),
        out_type=(
            jax.ShapeDtypeStruct((E,), jnp.float32),
            jax.ShapeDtypeStruct((2, NW, 16), jnp.float32),
        ),
        scratch_types=[
            pltpu.VMEM((2, BLK), jnp.int32),
            pltpu.VMEM((2, BLK), jnp.int32),
            pltpu.VMEM((2, BLK), jnp.float32),
            pltpu.VMEM((2 * BLK, D), jnp.float32),
            pltpu.VMEM((2 * BLK, D), jnp.float32),
            pltpu.VMEM((2, BLK), jnp.float32),
            pltpu.VMEM((2, 16), jnp.float32),
            pltpu.SemaphoreType.DMA,
            pltpu.SemaphoreType.DMA,
            pltpu.SemaphoreType.DMA,
            pltpu.SemaphoreType.DMA,
            pltpu.SemaphoreType.DMA,
            pltpu.SemaphoreType.DMA,
            pltpu.SemaphoreType.DMA,
        ],
    )
    def sc_kernel(src_hbm, dst_hbm, lab_hbm, tab_hbm, y_hbm, part_hbm,
                  idx_s, idx_d, lab_v, rows_s, rows_d, y_v, acc_v,
                  sem_s, sem_s2, sem_d, sem_d2, sem_i, sem_y0, sem_y1):
        wid = lax.axis_index("s") * NC + lax.axis_index("c")
        base = wid * EPW
        iota = lax.iota(jnp.int32, 16)
        zeros = jnp.zeros((16,), jnp.float32)
        zero16 = jnp.zeros((16,), jnp.int32)

        def fire_gathers(par):
            for j in range(NCHUNK):
                sl = pl.ds(j * CHUNK, CHUNK)
                roff = pl.ds(par * BLK + j * CHUNK, CHUNK)
                pltpu.async_copy(
                    tab_hbm.at[idx_s.at[par, sl]], rows_s.at[roff],
                    sem_s if j % 2 == 0 else sem_s2)
                pltpu.async_copy(
                    tab_hbm.at[idx_d.at[par, sl]], rows_d.at[roff],
                    sem_d if j % 2 == 0 else sem_d2)

        def fire_idx(par, k):
            off = pl.ds(base + k * BLK, BLK)
            pltpu.async_copy(src_hbm.at[off], idx_s.at[par], sem_i)
            pltpu.async_copy(dst_hbm.at[off], idx_d.at[par], sem_i)

        def drain_rows(par):
            del par
            pltpu.make_async_copy(
                tab_hbm.at[pl.ds(0, 3 * CHUNK)], rows_s.at[pl.ds(0, 3 * CHUNK)],
                sem_s).wait()
            pltpu.make_async_copy(
                tab_hbm.at[pl.ds(0, 2 * CHUNK)], rows_s.at[pl.ds(0, 2 * CHUNK)],
                sem_s2).wait()
            pltpu.make_async_copy(
                tab_hbm.at[pl.ds(0, 3 * CHUNK)], rows_d.at[pl.ds(0, 3 * CHUNK)],
                sem_d).wait()
            pltpu.make_async_copy(
                tab_hbm.at[pl.ds(0, 2 * CHUNK)], rows_d.at[pl.ds(0, 2 * CHUNK)],
                sem_d2).wait()

        def drain_idx():
            pltpu.make_async_copy(
                src_hbm.at[pl.ds(0, BLK)], idx_s.at[0], sem_i).wait()
            pltpu.make_async_copy(
                dst_hbm.at[pl.ds(0, BLK)], idx_d.at[0], sem_i).wait()
            pltpu.make_async_copy(
                lab_hbm.at[pl.ds(0, BLK)], lab_v.at[0], sem_i).wait()

        pltpu.sync_copy(src_hbm.at[pl.ds(base, BLK)], idx_s.at[0])
        pltpu.sync_copy(dst_hbm.at[pl.ds(base, BLK)], idx_d.at[0])
        pltpu.sync_copy(lab_hbm.at[pl.ds(base, BLK)], lab_v.at[0])
        fire_gathers(0)
        fire_idx(1, 1)
        pltpu.async_copy(lab_hbm.at[pl.ds(base + BLK, BLK)], lab_v.at[1],
                         sem_i)

        def block_body(k, acc):
            a_sse, a_nrm = acc
            par = k & 1
            nxt = 1 - par
            off = base + k * BLK

            drain_rows(par)

            @pl.when(k < NBLK - 1)
            def _():
                drain_idx()
                fire_gathers(nxt)

            @pl.when(k < NBLK - 2)
            def _():
                fire_idx(par, k + 2)

            @pl.when(jnp.logical_and(k >= 2, par == 0))
            def _():
                pltpu.make_async_copy(
                    y_v.at[0], y_hbm.at[pl.ds(base, BLK)], sem_y0).wait()

            @pl.when(jnp.logical_and(k >= 2, par == 1))
            def _():
                pltpu.make_async_copy(
                    y_v.at[1], y_hbm.at[pl.ds(base, BLK)], sem_y1).wait()

            def grp_body(g, acc):
                a_sse, a_nrm = acc
                row0 = g * 16
                ids = par * BLK + row0 + iota
                basev = ids * D + iota
                y = zeros
                n2 = zeros
                for d in range(D):
                    addr = jnp.bitwise_xor(basev, d)
                    a = plsc.load_gather(rows_s, [zero16, addr])
                    b = plsc.load_gather(rows_d, [zero16, addr])
                    y = y + a * b
                    n2 = n2 + a * a
                lab = lab_v[par, pl.ds(row0, 16)]
                y_v[par, pl.ds(row0, 16)] = y
                r = lab - y
                return (a_sse + r * r, a_nrm + _sqrt16(n2))

            acc = lax.fori_loop(0, NGRP, grp_body, (a_sse, a_nrm))

            @pl.when(par == 0)
            def _():
                pltpu.async_copy(y_v.at[0], y_hbm.at[pl.ds(off, BLK)], sem_y0)

            @pl.when(par == 1)
            def _():
                pltpu.async_copy(y_v.at[1], y_hbm.at[pl.ds(off, BLK)], sem_y1)

            @pl.when(k < NBLK - 2)
            def _():
                pltpu.async_copy(lab_hbm.at[pl.ds(off + 2 * BLK, BLK)],
                                 lab_v.at[par], sem_i)

            return acc

        a_sse, a_nrm = lax.fori_loop(0, NBLK, block_body, (zeros, zeros))

        pltpu.make_async_copy(
            y_v.at[0], y_hbm.at[pl.ds(base, BLK)], sem_y0).wait()
        pltpu.make_async_copy(
            y_v.at[1], y_hbm.at[pl.ds(base, BLK)], sem_y1).wait()

        acc_v[0, :] = a_sse
        acc_v[1, :] = a_nrm
        pltpu.sync_copy(acc_v.at[0], part_hbm.at[0, wid])
        pltpu.sync_copy(acc_v.at[1], part_hbm.at[1, wid])

    return sc_kernel


def _loss_body(p_ref, out_ref):
    s_sse = jnp.sum(p_ref[0:1, :])
    s_nrm = jnp.sum(p_ref[1:2, :])
    l = jnp.float32(E) * 0.5 * s_sse + jnp.float32(LAMBDA / 2.0) * s_nrm
    out_ref[...] = l.reshape(1, 1)


def kernel(edges, labels, embedding_state):
    edges = edges.astype(jnp.int32)
    src = edges[:, 0]
    dst = edges[:, 1]
    labels = labels.astype(jnp.float32)
    table = embedding_state.astype(jnp.float32)

    y, partials = _make_sc_kernel()(src, dst, labels, table)

    l_out = pl.pallas_call(
        _loss_body,
        out_shape=jax.ShapeDtypeStruct((1, 1), jnp.float32),
    )(partials.reshape(2, NW * 16))
    return (l_out[0, 0], y)
